# baseline (speedup 1.0000x reference)
import numpy as np

import jax
import jax.numpy as jnp
from jax.experimental import pallas as pl
from jax.experimental.pallas import tpu as pltpu
from jax.sharding import Mesh, PartitionSpec as P

_CS = 4096


def _css_body(p_ref, f_ref, o_ref, m_ref, r_ref):
    j = pl.program_id(1)

    @pl.when(j == 0)
    def _():
        p = p_ref[0]
        m = jnp.max(p, axis=1, keepdims=True)
        z = jnp.sum(jnp.exp(p - m), axis=1, keepdims=True)
        m_ref[...] = m
        r_ref[...] = 1.0 / z
        o_ref[...] = jnp.zeros_like(o_ref)

    cs = f_ref.shape[2]
    p_chunk = p_ref[0, :, pl.ds(pl.multiple_of(j * cs, cs), cs)]
    w = jnp.exp(p_chunk - m_ref[...]) * r_ref[...]
    acc = jax.lax.dot_general(
        f_ref[0], w, (((1,), (1,)), ((), ())),
        preferred_element_type=jnp.float32)
    o_ref[0] += acc


def _css_one_device(f, p):
    B, C, HW = f.shape
    K = p.shape[1]
    nchunk = HW // _CS
    return pl.pallas_call(
        _css_body,
        grid=(B, nchunk),
        in_specs=[
            pl.BlockSpec((1, K, HW), lambda b, j: (b, 0, 0)),
            pl.BlockSpec((1, C, _CS), lambda b, j: (b, 0, j)),
        ],
        out_specs=pl.BlockSpec((1, C, K), lambda b, j: (b, 0, 0)),
        out_shape=jax.ShapeDtypeStruct((B, C, K), jnp.float32),
        scratch_shapes=[
            pltpu.VMEM((K, 1), jnp.float32),
            pltpu.VMEM((K, 1), jnp.float32),
        ],
        compiler_params=pltpu.CompilerParams(
            dimension_semantics=("parallel", "arbitrary"),
            vmem_limit_bytes=48 * 1024 * 1024,
        ),
        name="css_softmax_pool",
    )(p, f)


def kernel(feats, probs):
    B, K, H, W = probs.shape
    C = feats.shape[1]
    HW = H * W
    f = feats.reshape(B, C, HW)
    p = probs.reshape(B, K, HW)
    devs = jax.devices()
    n_shards = 2 if (len(devs) >= 2 and B % 2 == 0) else 1
    if n_shards == 2:
        mesh = Mesh(np.array(devs[:2]), ("d",))
        out = jax.shard_map(
            _css_one_device,
            mesh=mesh,
            in_specs=(P("d"), P("d")),
            out_specs=P("d"),
            check_vma=False,
        )(f, p)
    else:
        out = _css_one_device(f, p)
    return out[..., None]


# speedup vs baseline: 1.9392x; 1.9392x over previous
import jax
import jax.numpy as jnp
from jax.experimental import pallas as pl
from jax.experimental.pallas import tpu as pltpu

_CB = 128


def _css_body(p_ref, f_ref, o_ref, w_ref):
    cb = pl.program_id(1)

    @pl.when(cb == 0)
    def _():
        p = p_ref[0]
        m = jnp.max(p, axis=1, keepdims=True)
        e = jnp.exp(p - m)
        z = jnp.sum(e, axis=1, keepdims=True)
        w_ref[...] = e * (1.0 / z)

    o_ref[0] = jax.lax.dot_general(
        f_ref[0], w_ref[...], (((1,), (1,)), ((), ())),
        preferred_element_type=jnp.float32)


def kernel(feats, probs):
    B, K, H, W = probs.shape
    C = feats.shape[1]
    HW = H * W
    f = feats.reshape(B, C, HW)
    p = probs.reshape(B, K, HW)
    out = pl.pallas_call(
        _css_body,
        grid=(B, C // _CB),
        in_specs=[
            pl.BlockSpec((1, K, HW), lambda b, cb: (b, 0, 0)),
            pl.BlockSpec((1, _CB, HW), lambda b, cb: (b, cb, 0)),
        ],
        out_specs=pl.BlockSpec((1, _CB, K), lambda b, cb: (b, cb, 0)),
        out_shape=jax.ShapeDtypeStruct((B, C, K), jnp.float32),
        scratch_shapes=[
            pltpu.VMEM((K, HW), jnp.float32),
        ],
        compiler_params=pltpu.CompilerParams(
            dimension_semantics=("parallel", "arbitrary"),
            vmem_limit_bytes=48 * 1024 * 1024,
        ),
        name="css_softmax_pool",
    )(p, f)
    return out[..., None]
